# Initial kernel scaffold; baseline (speedup 1.0000x reference)
import jax
import jax.numpy as jnp
from jax.experimental import pallas as pl


def kernel(x, edge_index, edge_attr, We1, be1, g1, bb1, Wn1, bn1, gn, bbn, Wm2, bm2, We2, be2, Wn2, bn2):
    raise NotImplementedError("write your pallas kernel here")


# speedup vs baseline: 5.4030x; 5.4030x over previous
import functools

import jax
import jax.numpy as jnp
from jax import lax
from jax.experimental import pallas as pl
from jax.experimental.pallas import tpu as pltpu
from jax.experimental.pallas import tpu_sc as plsc

_N = 50000
_E = 800000
_NC = 2
_NS = 16
_NW = _NC * _NS
_EPW = _E // _NW
_NP = 50048
_RPT = _NP // _NS

_CHA = 128
_NFULL_A = _EPW // _CHA
_TAIL_A = _EPW - _NFULL_A * _CHA

_CHB = 2000
_NFULL_B = _EPW // _CHB
_TAIL_B = _EPW - _NFULL_B * _CHB


def _sc_kernel_a(row_h, col_h, ea2_h, xaug_h, zeros_h,
                 outr0_h, outc0_h, outr1_h, outc1_h,
                 srow, scol, idx_r, idx_c, pay,
                 idx_rt, idx_ct, payt, sem):
  cid = lax.axis_index("c")
  sid = lax.axis_index("s")
  wid = sid * _NC + cid
  base = wid * _EPW

  zb = sid * _RPT
  pltpu.sync_copy(zeros_h.at[pl.ds(zb, _RPT)], srow.at[pl.ds(zb, _RPT)])
  pltpu.sync_copy(zeros_h.at[pl.ds(zb, _RPT)], scol.at[pl.ds(zb, _RPT)])
  plsc.subcore_barrier()

  def do_chunk(off, k, ir, ic, pv):
    pltpu.sync_copy(row_h.at[pl.ds(off, k)], ir)
    pltpu.sync_copy(col_h.at[pl.ds(off, k)], ic)
    pltpu.async_copy(xaug_h.at[ic], pv, sem).wait()
    pltpu.sync_copy(ea2_h.at[pl.ds(off, k), :], pv.at[:, pl.ds(4, 1)])
    pltpu.sync_copy(pv, srow.at[ir], add=True)
    pltpu.sync_copy(pv, scol.at[ic], add=True)

  def loop_body(j, carry):
    do_chunk(base + j * _CHA, _CHA, idx_r, idx_c, pay)
    return carry

  lax.fori_loop(0, _NFULL_A, loop_body, 0)
  do_chunk(base + _NFULL_A * _CHA, _TAIL_A, idx_rt, idx_ct, payt)

  plsc.subcore_barrier()

  @pl.when(cid == 0)
  def _():
    pltpu.sync_copy(srow.at[pl.ds(zb, _RPT)], outr0_h.at[pl.ds(zb, _RPT)])
    pltpu.sync_copy(scol.at[pl.ds(zb, _RPT)], outc0_h.at[pl.ds(zb, _RPT)])

  @pl.when(cid == 1)
  def _():
    pltpu.sync_copy(srow.at[pl.ds(zb, _RPT)], outr1_h.at[pl.ds(zb, _RPT)])
    pltpu.sync_copy(scol.at[pl.ds(zb, _RPT)], outc1_h.at[pl.ds(zb, _RPT)])


def _run_sc_a(row, col, ea2, xaug, zeros8):
  mesh = plsc.VectorSubcoreMesh(core_axis_name="c", subcore_axis_name="s")
  f = pl.kernel(
      _sc_kernel_a,
      out_type=[jax.ShapeDtypeStruct((_NP, 8), jnp.float32)] * 4,
      mesh=mesh,
      compiler_params=pltpu.CompilerParams(use_tc_tiling_on_sc=False),
      scratch_types=[
          pltpu.VMEM_SHARED((_NP, 8), jnp.float32),
          pltpu.VMEM_SHARED((_NP, 8), jnp.float32),
          pltpu.VMEM((_CHA,), jnp.int32),
          pltpu.VMEM((_CHA,), jnp.int32),
          pltpu.VMEM((_CHA, 8), jnp.float32),
          pltpu.VMEM((_TAIL_A,), jnp.int32),
          pltpu.VMEM((_TAIL_A,), jnp.int32),
          pltpu.VMEM((_TAIL_A, 8), jnp.float32),
          pltpu.SemaphoreType.DMA,
      ],
  )
  return f(row, col, ea2, xaug, zeros8)


def _sc_kernel_b(row_h, col_h, ea_h, u1_h, u2_h, r_h, t_h, e2_h,
                 u1v, u2v, idx_r, idx_c, ea_v, out_v, r_v, t_v):
  cid = lax.axis_index("c")
  sid = lax.axis_index("s")
  wid = sid * _NC + cid
  base = wid * _EPW

  pltpu.sync_copy(u1_h, u1v)
  pltpu.sync_copy(u2_h, u2v)
  pltpu.sync_copy(r_h, r_v)
  pltpu.sync_copy(t_h, t_v)
  rv = r_v[...]
  tv = t_v[...]
  iota16 = lax.iota(jnp.int32, 16)

  def do_chunk(off, k, n16, tail8):
    pltpu.sync_copy(row_h.at[pl.ds(off, k)], idx_r.at[pl.ds(0, k)])
    pltpu.sync_copy(col_h.at[pl.ds(off, k)], idx_c.at[pl.ds(0, k)])
    pltpu.sync_copy(ea_h.at[pl.ds(off, k)], ea_v.at[pl.ds(0, k)])

    def g_body(g, carry):
      ir = idx_r[pl.ds(g * 16, 16)]
      ic = idx_c[pl.ds(g * 16, 16)]
      ev = ea_v[pl.ds(g * 16, 16)]
      g1 = plsc.load_gather(u1v, [ir])
      g2 = plsc.load_gather(u2v, [ic])
      out_v[pl.ds(g * 16, 16)] = jnp.maximum(g1 + g2 + ev * rv + tv, 0.0)
      return carry

    lax.fori_loop(0, n16, g_body, 0)
    if tail8:
      gb = n16 * 16
      msk = iota16 < 8
      ir = idx_r[pl.ds(gb, 16)]
      ic = idx_c[pl.ds(gb, 16)]
      ev = ea_v[pl.ds(gb, 16)]
      g1 = plsc.load_gather(u1v, [ir], mask=msk)
      g2 = plsc.load_gather(u2v, [ic], mask=msk)
      out_v[pl.ds(gb, 16)] = jnp.maximum(g1 + g2 + ev * rv + tv, 0.0)
    pltpu.sync_copy(out_v.at[pl.ds(0, k)], e2_h.at[pl.ds(off, k)])

  def loop_body(j, carry):
    do_chunk(base + j * _CHB, _CHB, _CHB // 16, False)
    return carry

  lax.fori_loop(0, _NFULL_B, loop_body, 0)
  do_chunk(base + _NFULL_B * _CHB, _TAIL_B, _TAIL_B // 16, True)


def _run_sc_b(row, col, ea, u1, u2, rfull, tfull):
  mesh = plsc.VectorSubcoreMesh(core_axis_name="c", subcore_axis_name="s")
  f = pl.kernel(
      _sc_kernel_b,
      out_type=jax.ShapeDtypeStruct((_E,), jnp.float32),
      mesh=mesh,
      compiler_params=pltpu.CompilerParams(use_tc_tiling_on_sc=False,
                                           needs_layout_passes=False),
      scratch_types=[
          pltpu.VMEM((_N,), jnp.float32),
          pltpu.VMEM((_N,), jnp.float32),
          pltpu.VMEM((_CHB,), jnp.int32),
          pltpu.VMEM((_CHB,), jnp.int32),
          pltpu.VMEM((_CHB,), jnp.float32),
          pltpu.VMEM((_CHB,), jnp.float32),
          pltpu.VMEM((16,), jnp.float32),
          pltpu.VMEM((16,), jnp.float32),
      ],
  )
  return f(row, col, ea, u1, u2, rfull, tfull)


_R_TC = 2000
_G_TC = _N // _R_TC


def _tc_stats_kernel(x_ref, sr0, sr1, sc0, sc1, o1, o2, o3, o4, o5):
  @pl.when(pl.program_id(0) == 0)
  def _():
    o1[...] = jnp.zeros((4, 8), jnp.float32)
    o2[...] = jnp.zeros((4, 8), jnp.float32)
    o3[...] = jnp.zeros((4, 4), jnp.float32)
    o4[...] = jnp.zeros((4, 4), jnp.float32)
    o5[...] = jnp.zeros((1, 8), jnp.float32)

  xb = x_ref[...]
  srow = sr0[...] + sr1[...]
  scol = sc0[...] + sc1[...]
  cnt = srow[:, 5:6]
  cntc = scol[:, 5:6]
  dn = (((0,), (0,)), ((), ()))
  hi = lax.Precision.HIGHEST
  o1[...] += lax.dot_general(xb, srow, dn, precision=hi)
  o2[...] += lax.dot_general(xb, scol, dn, precision=hi)
  o3[...] += lax.dot_general(xb, xb * cnt, dn, precision=hi)
  o4[...] += lax.dot_general(xb, xb * cntc, dn, precision=hi)
  o5[...] += jnp.sum(srow, axis=0, keepdims=True)


def _run_tc_stats(x, SR0, SR1, SC0, SC1):
  return pl.pallas_call(
      _tc_stats_kernel,
      grid=(_G_TC,),
      in_specs=[
          pl.BlockSpec((_R_TC, 4), lambda i: (i, 0)),
          pl.BlockSpec((_R_TC, 8), lambda i: (i, 0)),
          pl.BlockSpec((_R_TC, 8), lambda i: (i, 0)),
          pl.BlockSpec((_R_TC, 8), lambda i: (i, 0)),
          pl.BlockSpec((_R_TC, 8), lambda i: (i, 0)),
      ],
      out_specs=[
          pl.BlockSpec((4, 8), lambda i: (0, 0)),
          pl.BlockSpec((4, 8), lambda i: (0, 0)),
          pl.BlockSpec((4, 4), lambda i: (0, 0)),
          pl.BlockSpec((4, 4), lambda i: (0, 0)),
          pl.BlockSpec((1, 8), lambda i: (0, 0)),
      ],
      out_shape=[
          jax.ShapeDtypeStruct((4, 8), jnp.float32),
          jax.ShapeDtypeStruct((4, 8), jnp.float32),
          jax.ShapeDtypeStruct((4, 4), jnp.float32),
          jax.ShapeDtypeStruct((4, 4), jnp.float32),
          jax.ShapeDtypeStruct((1, 8), jnp.float32),
      ],
  )(x, SR0, SR1, SC0, SC1)


def _tc_ea2_kernel(ea_ref, o_ref):
  v = ea_ref[...]
  o_ref[...] = jnp.sum(v * v).reshape(1, 1)


def _run_tc_ea2(ea2d):
  return pl.pallas_call(
      _tc_ea2_kernel,
      out_shape=jax.ShapeDtypeStruct((1, 1), jnp.float32),
  )(ea2d)


def _tc_node_kernel(x_ref, sr0, sr1, q_ref, ac_ref, w2x_ref, w2a_ref,
                    bm_ref, vp_ref, pq_ref, u_ref):
  xb = x_ref[...]
  srow = sr0[...] + sr1[...]
  cnt = srow[:, 5:6]
  m = jnp.maximum(cnt, 1.0)
  sx = srow[:, 0:4]
  sa = srow[:, 4:5]
  Q = q_ref[...]
  hi = lax.Precision.HIGHEST
  dot = lambda a, b: jnp.dot(a, b, precision=hi)
  aggpre = (dot(xb * cnt, Q[0:4]) + dot(sx, Q[4:8]) + dot(sa, Q[8:9])
            + dot(cnt, Q[9:10]))
  a2 = ac_ref[0:1, :]
  c2 = ac_ref[1:2, :]
  agg = (aggpre * a2) / m + (cnt / m) * c2
  x1 = jnp.maximum(dot(xb, w2x_ref[...]) + dot(agg, w2a_ref[...])
                   + bm_ref[...], 0.0)
  u_ref[...] = dot(x1, vp_ref[...]) + dot(xb, pq_ref[...])


def _run_tc_node(x, SR0, SR1, Qp, A2C2, W2X, W2A, BM1, VP, PQ):
  cst = lambda i: (0, 0)
  return pl.pallas_call(
      _tc_node_kernel,
      grid=(_G_TC,),
      in_specs=[
          pl.BlockSpec((_R_TC, 4), lambda i: (i, 0)),
          pl.BlockSpec((_R_TC, 8), lambda i: (i, 0)),
          pl.BlockSpec((_R_TC, 8), lambda i: (i, 0)),
          pl.BlockSpec((16, 128), cst),
          pl.BlockSpec((2, 128), cst),
          pl.BlockSpec((4, 64), cst),
          pl.BlockSpec((128, 64), cst),
          pl.BlockSpec((1, 64), cst),
          pl.BlockSpec((64, 2), cst),
          pl.BlockSpec((4, 2), cst),
      ],
      out_specs=pl.BlockSpec((_R_TC, 2), lambda i: (i, 0)),
      out_shape=jax.ShapeDtypeStruct((_N, 2), jnp.float32),
  )(x, SR0, SR1, Qp, A2C2, W2X, W2A, BM1, VP, PQ)


def kernel(x, edge_index, edge_attr, We1, be1, g1, bb1, Wn1, bn1, gn, bbn,
           Wm2, bm2, We2, be2, Wn2, bn2):
  E = float(_E)
  row = edge_index[0]
  col = edge_index[1]
  ea = edge_attr[:, 0]
  ones_col = jnp.zeros((_N, 4), jnp.float32).at[:, 1].set(1.0)
  xaug = jnp.concatenate([x, ones_col], axis=1)
  zeros8 = jnp.zeros((_NP, 8), jnp.float32)

  SR0, SC0, SR1, SC1 = _run_sc_a(row, col, edge_attr, xaug, zeros8)

  O1, O2, O3, O4, O5 = _run_tc_stats(x, SR0, SR1, SC0, SC1)
  sum_ea2 = _run_tc_ea2(ea.reshape(_E // 128, 128))[0, 0]

  M = jnp.zeros((10, 10), jnp.float32)
  M = M.at[0:4, 0:4].set(O3)
  M = M.at[4:8, 4:8].set(O4)
  M = M.at[0:4, 4:8].set(O1[:, 0:4])
  M = M.at[4:8, 0:4].set(O1[:, 0:4].T)
  M = M.at[0:4, 8].set(O1[:, 4])
  M = M.at[8, 0:4].set(O1[:, 4])
  M = M.at[0:4, 9].set(O1[:, 5])
  M = M.at[9, 0:4].set(O1[:, 5])
  M = M.at[4:8, 8].set(O2[:, 4])
  M = M.at[8, 4:8].set(O2[:, 4])
  M = M.at[4:8, 9].set(O2[:, 5])
  M = M.at[9, 4:8].set(O2[:, 5])
  M = M.at[8, 8].set(sum_ea2)
  M = M.at[8, 9].set(O5[0, 4])
  M = M.at[9, 8].set(O5[0, 4])
  M = M.at[9, 9].set(E)

  hi = lax.Precision.HIGHEST
  dot = lambda a, b: jnp.dot(a, b, precision=hi)

  W1aug = jnp.concatenate([We1, be1[None, :]], axis=0)
  sumz = M[:, 9]
  mean1 = dot(sumz, W1aug) / E
  m2_1 = jnp.sum(W1aug * dot(M, W1aug), axis=0) / E
  a1 = g1 * lax.rsqrt(m2_1 - mean1 * mean1 + 1e-5)
  c1 = bb1 - mean1 * a1

  Q = dot(W1aug, a1[:, None] * Wn1[4:88])
  Q = Q.at[4:8].add(Wn1[0:4])
  Q = Q.at[9].add(dot(c1, Wn1[4:88]) + bn1)

  mean_h = dot(sumz, Q) / E
  m2_h = jnp.sum(Q * dot(M, Q), axis=0) / E
  a2 = gn * lax.rsqrt(m2_h - mean_h * mean_h + 1e-5)
  c2 = bbn - mean_h * a2

  w_e1 = We2[128:212, 0]
  aw = a1 * w_e1
  pvec = dot(We1[0:4], aw)
  qvec = dot(We1[4:8], aw)
  rsc = dot(We1[8], aw)
  tsc = dot(be1, aw) + dot(c1, w_e1) + be2[0]

  Qp = jnp.zeros((16, 128), jnp.float32).at[0:10].set(Q)
  A2C2 = jnp.stack([a2, c2], axis=0)
  VP = jnp.stack([We2[0:64, 0], We2[64:128, 0]], axis=1)
  PQ = jnp.stack([pvec, qvec], axis=1)

  U = _run_tc_node(x, SR0, SR1, Qp, A2C2, Wm2[0:4], Wm2[4:132],
                   bm2[None, :], VP, PQ)
  u1 = U[:, 0] + 0.0
  u2 = U[:, 1] + 0.0
  rfull = jnp.full((16,), rsc, jnp.float32)
  tfull = jnp.full((16,), tsc, jnp.float32)

  e2 = _run_sc_b(row, col, ea, u1, u2, rfull, tfull)
  return e2[:, None]
